# Initial kernel scaffold; baseline (speedup 1.0000x reference)
import jax
import jax.numpy as jnp
from jax.experimental import pallas as pl


def kernel(x, patch_w, patch_b, pos_embed, in_proj_w, conv_w, conv_b, x_proj_w, dt_proj_w, dt_proj_b, A_log, D_param, out_proj_w, norm_w, norm_b):
    raise NotImplementedError("write your pallas kernel here")


# speedup vs baseline: 7.0506x; 7.0506x over previous
import jax
import jax.numpy as jnp
from jax.experimental import pallas as pl
from jax.experimental.pallas import tpu as pltpu

B = 8
C = 3
IMG = 224
P = 16
D = 768
NL = 12
DI = 2 * D
DS = 16
DTR = 48
N = (IMG // P) ** 2
DCONV = 4
L = N
CH = 8
NC = (L + CH - 1) // CH
LP = NC * CH

_F32 = jnp.float32


def _embed_body(xp_ref, pw_ref, pb_ref, pos_ref, out_ref):
    h = jnp.dot(xp_ref[0], pw_ref[...], preferred_element_type=_F32)
    out_ref[0] = h + pb_ref[...] + pos_ref[0]


def _layer_body(h_ref, inw_ref, cw_ref, cb_ref, xpw_ref, dtw_ref, dtb_ref,
                alog_ref, dpar_ref, ow_ref, out_ref,
                dt_s, dtx_s, bm_s, cm_s, ys_s):
    h = h_ref[0]
    xz = jnp.dot(h, inw_ref[...], preferred_element_type=_F32)
    xc = xz[:, :DI]
    z = xz[:, DI:]

    xpad = jnp.pad(xc, ((DCONV - 1, 0), (0, 0)))
    acc = cb_ref[...]
    for k in range(DCONV):
        acc = acc + xpad[k:k + L, :] * cw_ref[k:k + 1, :]
    xcs = acc * jax.nn.sigmoid(acc)

    proj = jnp.dot(xcs, xpw_ref[...], preferred_element_type=_F32)
    dt = jnp.dot(proj[:, :DTR], dtw_ref[...],
                 preferred_element_type=_F32) + dtb_ref[...]
    dt = jax.nn.softplus(dt)
    bm = proj[:, DTR:DTR + DS]
    cm = proj[:, DTR + DS:DTR + 2 * DS]

    pad_t = ((0, LP - L), (0, 0))
    dt_s[...] = jnp.pad(dt, pad_t).reshape(NC, CH, DI)
    dtx_s[...] = jnp.pad(dt * xcs, pad_t).reshape(NC, CH, DI)
    bm_s[...] = jnp.pad(bm, pad_t).reshape(NC, CH, DS)
    cm_s[...] = jnp.pad(cm, pad_t).reshape(NC, CH, DS)

    A = -jnp.exp(alog_ref[...])

    def chunk(c, hst):
        dtc = dt_s[pl.ds(c, 1)][0]
        dxc = dtx_s[pl.ds(c, 1)][0]
        bT = jnp.transpose(bm_s[pl.ds(c, 1)][0])
        cT = jnp.transpose(cm_s[pl.ds(c, 1)][0])
        ys_rows = []
        for s in range(CH):
            a = jnp.exp(A * dtc[s:s + 1, :])
            b = bT[:, s:s + 1] * dxc[s:s + 1, :]
            hst = a * hst + b
            ys_rows.append(jnp.sum(hst * cT[:, s:s + 1], axis=0, keepdims=True))
        ys_s[pl.ds(c, 1)] = jnp.concatenate(ys_rows, axis=0).reshape(1, CH, DI)
        return hst

    jax.lax.fori_loop(0, NC, chunk, jnp.zeros((DS, DI), _F32))

    ys = ys_s[...].reshape(LP, DI)[:L, :]
    y = ys + xcs * dpar_ref[...]
    y = y * (z * jax.nn.sigmoid(z))
    out_ref[0] = jnp.dot(y, ow_ref[...], preferred_element_type=_F32)


def _ln_body(h_ref, w_ref, b_ref, out_ref):
    h = h_ref[0]
    mu = jnp.mean(h, axis=1, keepdims=True)
    d = h - mu
    var = jnp.mean(d * d, axis=1, keepdims=True)
    out_ref[0] = d * jax.lax.rsqrt(var + 1e-5) * w_ref[...] + b_ref[...]


_CPARAMS = pltpu.CompilerParams(
    dimension_semantics=("parallel",),
    vmem_limit_bytes=100 * 1024 * 1024,
)


def _make_layer(Bz):
    return pl.pallas_call(
        _layer_body,
        grid=(Bz,),
        in_specs=[
            pl.BlockSpec((1, L, D), lambda b: (b, 0, 0)),
            pl.BlockSpec((D, 2 * DI), lambda b: (0, 0)),
            pl.BlockSpec((DCONV, DI), lambda b: (0, 0)),
            pl.BlockSpec((1, DI), lambda b: (0, 0)),
            pl.BlockSpec((DI, DTR + 2 * DS), lambda b: (0, 0)),
            pl.BlockSpec((DTR, DI), lambda b: (0, 0)),
            pl.BlockSpec((1, DI), lambda b: (0, 0)),
            pl.BlockSpec((DS, DI), lambda b: (0, 0)),
            pl.BlockSpec((1, DI), lambda b: (0, 0)),
            pl.BlockSpec((DI, D), lambda b: (0, 0)),
        ],
        out_specs=pl.BlockSpec((1, L, D), lambda b: (b, 0, 0)),
        out_shape=jax.ShapeDtypeStruct((Bz, L, D), _F32),
        scratch_shapes=[
            pltpu.VMEM((NC, CH, DI), _F32),
            pltpu.VMEM((NC, CH, DI), _F32),
            pltpu.VMEM((NC, CH, DS), _F32),
            pltpu.VMEM((NC, CH, DS), _F32),
            pltpu.VMEM((NC, CH, DI), _F32),
        ],
        compiler_params=_CPARAMS,
    )


def _run(x, patch_w, patch_b, pos_embed, in_proj_w, conv_w, conv_b,
         x_proj_w, dt_proj_w, dt_proj_b, A_log, D_param, out_proj_w,
         norm_w, norm_b):
    Bz = x.shape[0]
    Hn = IMG // P
    xp = x.reshape(Bz, C, Hn, P, Hn, P).transpose(0, 2, 4, 1, 3, 5)
    xp = xp.reshape(Bz, N, C * P * P)
    pw = patch_w.reshape(D, C * P * P).T
    pb = patch_b.reshape(1, D)
    inwT = in_proj_w.transpose(0, 2, 1)
    cwT = conv_w.transpose(0, 2, 1)
    cb2 = conv_b.reshape(NL, 1, DI)
    xpwT = x_proj_w.transpose(0, 2, 1)
    dtwT = dt_proj_w.transpose(0, 2, 1)
    dtb2 = dt_proj_b.reshape(NL, 1, DI)
    alogT = A_log.transpose(0, 2, 1)
    dpar2 = D_param.reshape(NL, 1, DI)
    owT = out_proj_w.transpose(0, 2, 1)

    cparams = _CPARAMS

    h = pl.pallas_call(
        _embed_body,
        grid=(Bz,),
        in_specs=[
            pl.BlockSpec((1, N, C * P * P), lambda b: (b, 0, 0)),
            pl.BlockSpec((C * P * P, D), lambda b: (0, 0)),
            pl.BlockSpec((1, D), lambda b: (0, 0)),
            pl.BlockSpec((1, N, D), lambda b: (0, 0, 0)),
        ],
        out_specs=pl.BlockSpec((1, N, D), lambda b: (b, 0, 0)),
        out_shape=jax.ShapeDtypeStruct((Bz, N, D), _F32),
        compiler_params=cparams,
    )(xp, pw, pb, pos_embed)

    layer = _make_layer(Bz)

    for l in range(NL):
        h = layer(h, inwT[l], cwT[l], cb2[l], xpwT[l], dtwT[l], dtb2[l],
                  alogT[l], dpar2[l], owT[l])

    out = pl.pallas_call(
        _ln_body,
        grid=(Bz,),
        in_specs=[
            pl.BlockSpec((1, L, D), lambda b: (b, 0, 0)),
            pl.BlockSpec((1, D), lambda b: (0, 0)),
            pl.BlockSpec((1, D), lambda b: (0, 0)),
        ],
        out_specs=pl.BlockSpec((1, L, D), lambda b: (b, 0, 0)),
        out_shape=jax.ShapeDtypeStruct((Bz, L, D), _F32),
        compiler_params=cparams,
    )(h, norm_w.reshape(1, D), norm_b.reshape(1, D))
    return out


def kernel(x, patch_w, patch_b, pos_embed, in_proj_w, conv_w, conv_b,
           x_proj_w, dt_proj_w, dt_proj_b, A_log, D_param, out_proj_w,
           norm_w, norm_b):
    return _run(x, patch_w, patch_b, pos_embed, in_proj_w, conv_w, conv_b,
                x_proj_w, dt_proj_w, dt_proj_b, A_log, D_param, out_proj_w,
                norm_w, norm_b)


# speedup vs baseline: 7.2936x; 1.0345x over previous
import jax
import jax.numpy as jnp
from jax.experimental import pallas as pl
from jax.experimental.pallas import tpu as pltpu

B = 8
C = 3
IMG = 224
P = 16
D = 768
NL = 12
DI = 2 * D
DS = 16
DTR = 48
N = (IMG // P) ** 2
DCONV = 4
L = N
CH = 8
NC = (L + CH - 1) // CH
LP = NC * CH

_F32 = jnp.float32


def _embed_body(xp_ref, pw_ref, pb_ref, pos_ref, out_ref):
    h = jnp.dot(xp_ref[0], pw_ref[...], preferred_element_type=_F32)
    out_ref[0] = h + pb_ref[...] + pos_ref[0]


_CONTRACT_LAST = (((1,), (1,)), ((), ()))


def _layer_body(h_ref, inw_ref, cw_ref, cb_ref, xpw_ref, dtw_ref, dtb_ref,
                alog_ref, dpar_ref, ow_ref, out_ref,
                dt_s, dtx_s, bm_s, cm_s, ys_s):
    h = h_ref[0].astype(jnp.bfloat16)
    xz = jax.lax.dot_general(h, inw_ref[...], _CONTRACT_LAST,
                             preferred_element_type=_F32)
    xc = xz[:, :DI]
    z = xz[:, DI:]

    xpad = jnp.pad(xc, ((DCONV - 1, 0), (0, 0)))
    acc = cb_ref[...]
    for k in range(DCONV):
        acc = acc + xpad[k:k + L, :] * cw_ref[k:k + 1, :]
    xcs = acc * jax.nn.sigmoid(acc)

    proj = jnp.dot(xcs.astype(jnp.bfloat16), xpw_ref[...],
                   preferred_element_type=_F32)
    dt = jnp.dot(proj[:, :DTR].astype(jnp.bfloat16), dtw_ref[...],
                 preferred_element_type=_F32) + dtb_ref[...]
    dt = jax.nn.softplus(dt)
    bm = proj[:, DTR:DTR + DS]
    cm = proj[:, DTR + DS:DTR + 2 * DS]

    pad_t = ((0, LP - L), (0, 0))
    dt_s[...] = jnp.pad(dt, pad_t).reshape(NC, CH, DI)
    dtx_s[...] = jnp.pad(dt * xcs, pad_t).reshape(NC, CH, DI)
    bm_s[...] = jnp.pad(bm, pad_t).reshape(NC, CH, DS)
    cm_s[...] = jnp.pad(cm, pad_t).reshape(NC, CH, DS)

    A = -jnp.exp(alog_ref[...])

    def chunk(c, hst):
        dtc = dt_s[pl.ds(c, 1)][0]
        dxc = dtx_s[pl.ds(c, 1)][0]
        bT = jnp.transpose(bm_s[pl.ds(c, 1)][0])
        cT = jnp.transpose(cm_s[pl.ds(c, 1)][0])
        ys_rows = []
        for s in range(CH):
            a = jnp.exp(A * dtc[s:s + 1, :])
            b = bT[:, s:s + 1] * dxc[s:s + 1, :]
            hst = a * hst + b
            ys_rows.append(jnp.sum(hst * cT[:, s:s + 1], axis=0, keepdims=True))
        ys_s[pl.ds(c, 1)] = jnp.concatenate(ys_rows, axis=0).reshape(1, CH, DI)
        return hst

    jax.lax.fori_loop(0, NC, chunk, jnp.zeros((DS, DI), _F32))

    ys = ys_s[...].reshape(LP, DI)[:L, :]
    y = ys + xcs * dpar_ref[...]
    y = y * (z * jax.nn.sigmoid(z))
    out_ref[0] = jax.lax.dot_general(y.astype(jnp.bfloat16), ow_ref[...],
                                     _CONTRACT_LAST,
                                     preferred_element_type=_F32)


def _ln_body(h_ref, w_ref, b_ref, out_ref):
    h = h_ref[0]
    mu = jnp.mean(h, axis=1, keepdims=True)
    d = h - mu
    var = jnp.mean(d * d, axis=1, keepdims=True)
    out_ref[0] = d * jax.lax.rsqrt(var + 1e-5) * w_ref[...] + b_ref[...]


_CPARAMS = pltpu.CompilerParams(
    dimension_semantics=("parallel",),
    vmem_limit_bytes=100 * 1024 * 1024,
)


def _make_layer(Bz):
    return pl.pallas_call(
        _layer_body,
        grid=(Bz,),
        in_specs=[
            pl.BlockSpec((1, L, D), lambda b: (b, 0, 0)),
            pl.BlockSpec((2 * DI, D), lambda b: (0, 0)),
            pl.BlockSpec((DCONV, DI), lambda b: (0, 0)),
            pl.BlockSpec((1, DI), lambda b: (0, 0)),
            pl.BlockSpec((DI, DTR + 2 * DS), lambda b: (0, 0)),
            pl.BlockSpec((DTR, DI), lambda b: (0, 0)),
            pl.BlockSpec((1, DI), lambda b: (0, 0)),
            pl.BlockSpec((DS, DI), lambda b: (0, 0)),
            pl.BlockSpec((1, DI), lambda b: (0, 0)),
            pl.BlockSpec((D, DI), lambda b: (0, 0)),
        ],
        out_specs=pl.BlockSpec((1, L, D), lambda b: (b, 0, 0)),
        out_shape=jax.ShapeDtypeStruct((Bz, L, D), _F32),
        scratch_shapes=[
            pltpu.VMEM((NC, CH, DI), _F32),
            pltpu.VMEM((NC, CH, DI), _F32),
            pltpu.VMEM((NC, CH, DS), _F32),
            pltpu.VMEM((NC, CH, DS), _F32),
            pltpu.VMEM((NC, CH, DI), _F32),
        ],
        compiler_params=_CPARAMS,
    )


def _run(x, patch_w, patch_b, pos_embed, in_proj_w, conv_w, conv_b,
         x_proj_w, dt_proj_w, dt_proj_b, A_log, D_param, out_proj_w,
         norm_w, norm_b):
    Bz = x.shape[0]
    Hn = IMG // P
    xp = x.reshape(Bz, C, Hn, P, Hn, P).transpose(0, 2, 4, 1, 3, 5)
    xp = xp.reshape(Bz, N, C * P * P).astype(jnp.bfloat16)
    pw = patch_w.reshape(D, C * P * P).T.astype(jnp.bfloat16)
    pb = patch_b.reshape(1, D)
    inw16 = in_proj_w.astype(jnp.bfloat16)
    cwT = conv_w.transpose(0, 2, 1)
    cb2 = conv_b.reshape(NL, 1, DI)
    xpwT = x_proj_w.transpose(0, 2, 1).astype(jnp.bfloat16)
    dtwT = dt_proj_w.transpose(0, 2, 1).astype(jnp.bfloat16)
    dtb2 = dt_proj_b.reshape(NL, 1, DI)
    alogT = A_log.transpose(0, 2, 1)
    dpar2 = D_param.reshape(NL, 1, DI)
    ow16 = out_proj_w.astype(jnp.bfloat16)

    cparams = _CPARAMS

    h = pl.pallas_call(
        _embed_body,
        grid=(Bz,),
        in_specs=[
            pl.BlockSpec((1, N, C * P * P), lambda b: (b, 0, 0)),
            pl.BlockSpec((C * P * P, D), lambda b: (0, 0)),
            pl.BlockSpec((1, D), lambda b: (0, 0)),
            pl.BlockSpec((1, N, D), lambda b: (0, 0, 0)),
        ],
        out_specs=pl.BlockSpec((1, N, D), lambda b: (b, 0, 0)),
        out_shape=jax.ShapeDtypeStruct((Bz, N, D), _F32),
        compiler_params=cparams,
    )(xp, pw, pb, pos_embed)

    layer = _make_layer(Bz)

    for l in range(NL):
        h = layer(h, inw16[l], cwT[l], cb2[l], xpwT[l], dtwT[l], dtb2[l],
                  alogT[l], dpar2[l], ow16[l])

    out = pl.pallas_call(
        _ln_body,
        grid=(Bz,),
        in_specs=[
            pl.BlockSpec((1, L, D), lambda b: (b, 0, 0)),
            pl.BlockSpec((1, D), lambda b: (0, 0)),
            pl.BlockSpec((1, D), lambda b: (0, 0)),
        ],
        out_specs=pl.BlockSpec((1, L, D), lambda b: (b, 0, 0)),
        out_shape=jax.ShapeDtypeStruct((Bz, L, D), _F32),
        compiler_params=cparams,
    )(h, norm_w.reshape(1, D), norm_b.reshape(1, D))
    return out


def kernel(x, patch_w, patch_b, pos_embed, in_proj_w, conv_w, conv_b,
           x_proj_w, dt_proj_w, dt_proj_b, A_log, D_param, out_proj_w,
           norm_w, norm_b):
    return _run(x, patch_w, patch_b, pos_embed, in_proj_w, conv_w, conv_b,
                x_proj_w, dt_proj_w, dt_proj_b, A_log, D_param, out_proj_w,
                norm_w, norm_b)


# speedup vs baseline: 7.2982x; 1.0006x over previous
import jax
import jax.numpy as jnp
from jax.experimental import pallas as pl
from jax.experimental.pallas import tpu as pltpu

B = 8
C = 3
IMG = 224
P = 16
D = 768
NL = 12
DI = 2 * D
DS = 16
DTR = 48
N = (IMG // P) ** 2
DCONV = 4
L = N
CH = 8
NC = (L + CH - 1) // CH
LP = NC * CH

_F32 = jnp.float32


def _embed_body(xp_ref, pw_ref, pb_ref, pos_ref, out_ref):
    h = jnp.dot(xp_ref[0], pw_ref[...], preferred_element_type=_F32)
    out_ref[0] = h + pb_ref[...] + pos_ref[0]


_CONTRACT_LAST = (((1,), (1,)), ((), ()))


def _layer_body(h_ref, inw_ref, cw_ref, cb_ref, xpw_ref, dtw_ref, dtb_ref,
                alog_ref, dpar_ref, ow_ref, out_ref,
                cv_s, dt_s, dtx_s, bm_s, cm_s, ys_s):
    h = h_ref[0].astype(jnp.bfloat16)
    xz = jax.lax.dot_general(h, inw_ref[...], _CONTRACT_LAST,
                             preferred_element_type=_F32)
    xc = xz[:, :DI]
    z = xz[:, DI:]

    cv_s[:DCONV - 1, :] = jnp.zeros((DCONV - 1, DI), _F32)
    cv_s[DCONV - 1:, :] = xc
    acc = cb_ref[...]
    for k in range(DCONV):
        acc = acc + cv_s[k:k + L, :] * cw_ref[k:k + 1, :]
    xcs = acc * jax.nn.sigmoid(acc)

    proj = jnp.dot(xcs.astype(jnp.bfloat16), xpw_ref[...],
                   preferred_element_type=_F32)
    dt = jnp.dot(proj[:, :DTR].astype(jnp.bfloat16), dtw_ref[...],
                 preferred_element_type=_F32) + dtb_ref[...]
    dt = jax.nn.softplus(dt)
    bm = proj[:, DTR:DTR + DS]
    cm = proj[:, DTR + DS:DTR + 2 * DS]

    pad_t = ((0, LP - L), (0, 0))
    dt_s[...] = jnp.pad(dt, pad_t)
    dtx_s[...] = jnp.pad(dt * xcs, pad_t)
    bm_s[...] = jnp.pad(bm, pad_t).reshape(NC, CH, DS)
    cm_s[...] = jnp.pad(cm, pad_t).reshape(NC, CH, DS)

    A = -jnp.exp(alog_ref[...])
    lane = jax.lax.broadcasted_iota(jnp.int32, (CH, CH * DS), 1)
    row = jax.lax.broadcasted_iota(jnp.int32, (CH, CH * DS), 0)
    cmask = (lane // DS) == row

    def chunk(c, hst):
        base = c * CH
        bT = jnp.transpose(bm_s[pl.ds(c, 1)][0])
        cc = cm_s[pl.ds(c, 1)][0]
        cblk = jnp.where(cmask, jnp.concatenate([cc] * CH, axis=1), 0.0)
        hs = []
        for s in range(CH):
            dtb = jnp.broadcast_to(dt_s[pl.ds(base + s, 1)], (DS, DI))
            dxb = jnp.broadcast_to(dtx_s[pl.ds(base + s, 1)], (DS, DI))
            a = jnp.exp(A * dtb)
            hst = a * hst + bT[:, s:s + 1] * dxb
            hs.append(hst)
        Hc = jnp.concatenate(hs, axis=0)
        Y = jnp.dot(cblk, Hc, preferred_element_type=_F32)
        ys_s[pl.ds(c, 1)] = Y.reshape(1, CH, DI)
        return hst

    jax.lax.fori_loop(0, NC, chunk, jnp.zeros((DS, DI), _F32))

    ys = ys_s[...].reshape(LP, DI)[:L, :]
    y = ys + xcs * dpar_ref[...]
    y = y * (z * jax.nn.sigmoid(z))
    out_ref[0] = jax.lax.dot_general(y.astype(jnp.bfloat16), ow_ref[...],
                                     _CONTRACT_LAST,
                                     preferred_element_type=_F32)


def _ln_body(h_ref, w_ref, b_ref, out_ref):
    h = h_ref[0]
    mu = jnp.mean(h, axis=1, keepdims=True)
    d = h - mu
    var = jnp.mean(d * d, axis=1, keepdims=True)
    out_ref[0] = d * jax.lax.rsqrt(var + 1e-5) * w_ref[...] + b_ref[...]


_CPARAMS = pltpu.CompilerParams(
    dimension_semantics=("parallel",),
    vmem_limit_bytes=100 * 1024 * 1024,
)


def _make_layer(Bz):
    return pl.pallas_call(
        _layer_body,
        grid=(Bz,),
        in_specs=[
            pl.BlockSpec((1, L, D), lambda b: (b, 0, 0)),
            pl.BlockSpec((2 * DI, D), lambda b: (0, 0)),
            pl.BlockSpec((DCONV, DI), lambda b: (0, 0)),
            pl.BlockSpec((1, DI), lambda b: (0, 0)),
            pl.BlockSpec((DI, DTR + 2 * DS), lambda b: (0, 0)),
            pl.BlockSpec((DTR, DI), lambda b: (0, 0)),
            pl.BlockSpec((1, DI), lambda b: (0, 0)),
            pl.BlockSpec((DS, DI), lambda b: (0, 0)),
            pl.BlockSpec((1, DI), lambda b: (0, 0)),
            pl.BlockSpec((D, DI), lambda b: (0, 0)),
        ],
        out_specs=pl.BlockSpec((1, L, D), lambda b: (b, 0, 0)),
        out_shape=jax.ShapeDtypeStruct((Bz, L, D), _F32),
        scratch_shapes=[
            pltpu.VMEM((DCONV - 1 + L, DI), _F32),
            pltpu.VMEM((LP, DI), _F32),
            pltpu.VMEM((LP, DI), _F32),
            pltpu.VMEM((NC, CH, DS), _F32),
            pltpu.VMEM((NC, CH, DS), _F32),
            pltpu.VMEM((NC, CH, DI), _F32),
        ],
        compiler_params=_CPARAMS,
    )


def _run(x, patch_w, patch_b, pos_embed, in_proj_w, conv_w, conv_b,
         x_proj_w, dt_proj_w, dt_proj_b, A_log, D_param, out_proj_w,
         norm_w, norm_b):
    Bz = x.shape[0]
    Hn = IMG // P
    xp = x.reshape(Bz, C, Hn, P, Hn, P).transpose(0, 2, 4, 1, 3, 5)
    xp = xp.reshape(Bz, N, C * P * P).astype(jnp.bfloat16)
    pw = patch_w.reshape(D, C * P * P).T.astype(jnp.bfloat16)
    pb = patch_b.reshape(1, D)
    inw16 = in_proj_w.astype(jnp.bfloat16)
    cwT = conv_w.transpose(0, 2, 1)
    cb2 = conv_b.reshape(NL, 1, DI)
    xpwT = x_proj_w.transpose(0, 2, 1).astype(jnp.bfloat16)
    dtwT = dt_proj_w.transpose(0, 2, 1).astype(jnp.bfloat16)
    dtb2 = dt_proj_b.reshape(NL, 1, DI)
    alogT = A_log.transpose(0, 2, 1)
    dpar2 = D_param.reshape(NL, 1, DI)
    ow16 = out_proj_w.astype(jnp.bfloat16)

    cparams = _CPARAMS

    h = pl.pallas_call(
        _embed_body,
        grid=(Bz,),
        in_specs=[
            pl.BlockSpec((1, N, C * P * P), lambda b: (b, 0, 0)),
            pl.BlockSpec((C * P * P, D), lambda b: (0, 0)),
            pl.BlockSpec((1, D), lambda b: (0, 0)),
            pl.BlockSpec((1, N, D), lambda b: (0, 0, 0)),
        ],
        out_specs=pl.BlockSpec((1, N, D), lambda b: (b, 0, 0)),
        out_shape=jax.ShapeDtypeStruct((Bz, N, D), _F32),
        compiler_params=cparams,
    )(xp, pw, pb, pos_embed)

    layer = _make_layer(Bz)

    for l in range(NL):
        h = layer(h, inw16[l], cwT[l], cb2[l], xpwT[l], dtwT[l], dtb2[l],
                  alogT[l], dpar2[l], ow16[l])

    out = pl.pallas_call(
        _ln_body,
        grid=(Bz,),
        in_specs=[
            pl.BlockSpec((1, L, D), lambda b: (b, 0, 0)),
            pl.BlockSpec((1, D), lambda b: (0, 0)),
            pl.BlockSpec((1, D), lambda b: (0, 0)),
        ],
        out_specs=pl.BlockSpec((1, L, D), lambda b: (b, 0, 0)),
        out_shape=jax.ShapeDtypeStruct((Bz, L, D), _F32),
        compiler_params=cparams,
    )(h, norm_w.reshape(1, D), norm_b.reshape(1, D))
    return out


def kernel(x, patch_w, patch_b, pos_embed, in_proj_w, conv_w, conv_b,
           x_proj_w, dt_proj_w, dt_proj_b, A_log, D_param, out_proj_w,
           norm_w, norm_b):
    return _run(x, patch_w, patch_b, pos_embed, in_proj_w, conv_w, conv_b,
                x_proj_w, dt_proj_w, dt_proj_b, A_log, D_param, out_proj_w,
                norm_w, norm_b)


# speedup vs baseline: 9.5179x; 1.3041x over previous
import jax
import jax.numpy as jnp
from jax.experimental import pallas as pl
from jax.experimental.pallas import tpu as pltpu

B = 8
C = 3
IMG = 224
P = 16
D = 768
NL = 12
DI = 2 * D
DS = 16
DTR = 48
N = (IMG // P) ** 2
DCONV = 4
L = N
CH = 8
NC = (L + CH - 1) // CH
LP = NC * CH

_F32 = jnp.float32


def _embed_body(xp_ref, pw_ref, pb_ref, pos_ref, out_ref):
    h = jnp.dot(xp_ref[0], pw_ref[...], preferred_element_type=_F32)
    out_ref[0] = h + pb_ref[...] + pos_ref[0]


_CONTRACT_LAST = (((1,), (1,)), ((), ()))


def _layer_body(h_ref, inw_ref, cw_ref, cb_ref, xpw_ref, dtw_ref, dtb_ref,
                alog_ref, dpar_ref, ow_ref, out_ref,
                cv_s, dt_s, dtx_s, bm_s, cm_s, ys_s):
    h = h_ref[0].astype(jnp.bfloat16)
    xz = jax.lax.dot_general(h, inw_ref[...], _CONTRACT_LAST,
                             preferred_element_type=_F32)
    xc = xz[:, :DI]
    z = xz[:, DI:]

    cv_s[:DCONV - 1, :] = jnp.zeros((DCONV - 1, DI), _F32)
    cv_s[DCONV - 1:, :] = xc
    acc = cb_ref[...]
    for k in range(DCONV):
        acc = acc + cv_s[k:k + L, :] * cw_ref[k:k + 1, :]
    xcs = acc * jax.nn.sigmoid(acc)

    proj = jnp.dot(xcs.astype(jnp.bfloat16), xpw_ref[...],
                   preferred_element_type=_F32)
    dt = jnp.dot(proj[:, :DTR].astype(jnp.bfloat16), dtw_ref[...],
                 preferred_element_type=_F32) + dtb_ref[...]
    dt = jax.nn.softplus(dt)
    bm = proj[:, DTR:DTR + DS]
    cm = proj[:, DTR + DS:DTR + 2 * DS]

    pad_t = ((0, LP - L), (0, 0))
    dt_s[...] = jnp.pad(dt, pad_t)
    dtx_s[...] = jnp.pad(dt * xcs, pad_t)
    bm_s[...] = jnp.pad(bm, pad_t).reshape(NC, CH, DS)
    cm_s[...] = jnp.pad(cm, pad_t).reshape(NC, CH, DS)

    A = -jnp.exp(alog_ref[...])
    lane = jax.lax.broadcasted_iota(jnp.int32, (CH, CH * DS), 1)
    row = jax.lax.broadcasted_iota(jnp.int32, (CH, CH * DS), 0)
    cmask = (lane // DS) == row

    def chunk(c, hst):
        base = c * CH
        bT = jnp.transpose(bm_s[pl.ds(c, 1)][0])
        cc = cm_s[pl.ds(c, 1)][0]
        cblk = jnp.where(cmask, jnp.concatenate([cc] * CH, axis=1), 0.0)
        hs = []
        for s in range(CH):
            dtb = jnp.broadcast_to(dt_s[pl.ds(base + s, 1)], (DS, DI))
            dxb = jnp.broadcast_to(dtx_s[pl.ds(base + s, 1)], (DS, DI))
            a = jnp.exp(A * dtb)
            hst = a * hst + bT[:, s:s + 1] * dxb
            hs.append(hst)
        Hc = jnp.concatenate(hs, axis=0)
        Y = jnp.dot(cblk, Hc, preferred_element_type=_F32)
        ys_s[pl.ds(c, 1)] = Y.reshape(1, CH, DI)
        return hst

    jax.lax.fori_loop(0, NC, chunk, jnp.zeros((DS, DI), _F32), unroll=5)

    ys = ys_s[...].reshape(LP, DI)[:L, :]
    y = ys + xcs * dpar_ref[...]
    y = y * (z * jax.nn.sigmoid(z))
    out_ref[0] = jax.lax.dot_general(y.astype(jnp.bfloat16), ow_ref[...],
                                     _CONTRACT_LAST,
                                     preferred_element_type=_F32)


def _ln_body(h_ref, w_ref, b_ref, out_ref):
    h = h_ref[0]
    mu = jnp.mean(h, axis=1, keepdims=True)
    d = h - mu
    var = jnp.mean(d * d, axis=1, keepdims=True)
    out_ref[0] = d * jax.lax.rsqrt(var + 1e-5) * w_ref[...] + b_ref[...]


_CPARAMS = pltpu.CompilerParams(
    dimension_semantics=("parallel",),
    vmem_limit_bytes=100 * 1024 * 1024,
)


def _make_layer(Bz):
    return pl.pallas_call(
        _layer_body,
        grid=(Bz,),
        in_specs=[
            pl.BlockSpec((1, L, D), lambda b: (b, 0, 0)),
            pl.BlockSpec((2 * DI, D), lambda b: (0, 0)),
            pl.BlockSpec((DCONV, DI), lambda b: (0, 0)),
            pl.BlockSpec((1, DI), lambda b: (0, 0)),
            pl.BlockSpec((DI, DTR + 2 * DS), lambda b: (0, 0)),
            pl.BlockSpec((DTR, DI), lambda b: (0, 0)),
            pl.BlockSpec((1, DI), lambda b: (0, 0)),
            pl.BlockSpec((DS, DI), lambda b: (0, 0)),
            pl.BlockSpec((1, DI), lambda b: (0, 0)),
            pl.BlockSpec((D, DI), lambda b: (0, 0)),
        ],
        out_specs=pl.BlockSpec((1, L, D), lambda b: (b, 0, 0)),
        out_shape=jax.ShapeDtypeStruct((Bz, L, D), _F32),
        scratch_shapes=[
            pltpu.VMEM((DCONV - 1 + L, DI), _F32),
            pltpu.VMEM((LP, DI), _F32),
            pltpu.VMEM((LP, DI), _F32),
            pltpu.VMEM((NC, CH, DS), _F32),
            pltpu.VMEM((NC, CH, DS), _F32),
            pltpu.VMEM((NC, CH, DI), _F32),
        ],
        compiler_params=_CPARAMS,
    )


def _run(x, patch_w, patch_b, pos_embed, in_proj_w, conv_w, conv_b,
         x_proj_w, dt_proj_w, dt_proj_b, A_log, D_param, out_proj_w,
         norm_w, norm_b):
    Bz = x.shape[0]
    Hn = IMG // P
    xp = x.reshape(Bz, C, Hn, P, Hn, P).transpose(0, 2, 4, 1, 3, 5)
    xp = xp.reshape(Bz, N, C * P * P).astype(jnp.bfloat16)
    pw = patch_w.reshape(D, C * P * P).T.astype(jnp.bfloat16)
    pb = patch_b.reshape(1, D)
    inw16 = in_proj_w.astype(jnp.bfloat16)
    cwT = conv_w.transpose(0, 2, 1)
    cb2 = conv_b.reshape(NL, 1, DI)
    xpwT = x_proj_w.transpose(0, 2, 1).astype(jnp.bfloat16)
    dtwT = dt_proj_w.transpose(0, 2, 1).astype(jnp.bfloat16)
    dtb2 = dt_proj_b.reshape(NL, 1, DI)
    alogT = A_log.transpose(0, 2, 1)
    dpar2 = D_param.reshape(NL, 1, DI)
    ow16 = out_proj_w.astype(jnp.bfloat16)

    cparams = _CPARAMS

    h = pl.pallas_call(
        _embed_body,
        grid=(Bz,),
        in_specs=[
            pl.BlockSpec((1, N, C * P * P), lambda b: (b, 0, 0)),
            pl.BlockSpec((C * P * P, D), lambda b: (0, 0)),
            pl.BlockSpec((1, D), lambda b: (0, 0)),
            pl.BlockSpec((1, N, D), lambda b: (0, 0, 0)),
        ],
        out_specs=pl.BlockSpec((1, N, D), lambda b: (b, 0, 0)),
        out_shape=jax.ShapeDtypeStruct((Bz, N, D), _F32),
        compiler_params=cparams,
    )(xp, pw, pb, pos_embed)

    layer = _make_layer(Bz)

    for l in range(NL):
        h = layer(h, inw16[l], cwT[l], cb2[l], xpwT[l], dtwT[l], dtb2[l],
                  alogT[l], dpar2[l], ow16[l])

    out = pl.pallas_call(
        _ln_body,
        grid=(Bz,),
        in_specs=[
            pl.BlockSpec((1, L, D), lambda b: (b, 0, 0)),
            pl.BlockSpec((1, D), lambda b: (0, 0)),
            pl.BlockSpec((1, D), lambda b: (0, 0)),
        ],
        out_specs=pl.BlockSpec((1, L, D), lambda b: (b, 0, 0)),
        out_shape=jax.ShapeDtypeStruct((Bz, L, D), _F32),
        compiler_params=cparams,
    )(h, norm_w.reshape(1, D), norm_b.reshape(1, D))
    return out


def kernel(x, patch_w, patch_b, pos_embed, in_proj_w, conv_w, conv_b,
           x_proj_w, dt_proj_w, dt_proj_b, A_log, D_param, out_proj_w,
           norm_w, norm_b):
    return _run(x, patch_w, patch_b, pos_embed, in_proj_w, conv_w, conv_b,
                x_proj_w, dt_proj_w, dt_proj_b, A_log, D_param, out_proj_w,
                norm_w, norm_b)


# speedup vs baseline: 9.9220x; 1.0425x over previous
import jax
import jax.numpy as jnp
from jax.experimental import pallas as pl
from jax.experimental.pallas import tpu as pltpu

B = 8
C = 3
IMG = 224
P = 16
D = 768
NL = 12
DI = 2 * D
DS = 16
DTR = 48
N = (IMG // P) ** 2
DCONV = 4
L = N
CH = 8
NC = (L + CH - 1) // CH
LP = NC * CH

_F32 = jnp.float32


def _embed_body(xp_ref, pw_ref, pb_ref, pos_ref, out_ref):
    h = jnp.dot(xp_ref[0], pw_ref[...], preferred_element_type=_F32)
    out_ref[0] = h + pb_ref[...] + pos_ref[0]


_CONTRACT_LAST = (((1,), (1,)), ((), ()))


def _layer_body(h_ref, inw_ref, cw_ref, cb_ref, xpw_ref, dtw_ref, dtb_ref,
                alog_ref, dpar_ref, ow_ref, out_ref,
                cv_s, dt_s, dtx_s, bm_s, cm_s, ys_s):
    h = h_ref[0].astype(jnp.bfloat16)
    xz = jax.lax.dot_general(h, inw_ref[...], _CONTRACT_LAST,
                             preferred_element_type=_F32)
    xc = xz[:, :DI]
    z = xz[:, DI:]

    cv_s[:DCONV - 1, :] = jnp.zeros((DCONV - 1, DI), _F32)
    cv_s[DCONV - 1:, :] = xc
    acc = cb_ref[...]
    for k in range(DCONV):
        acc = acc + cv_s[k:k + L, :] * cw_ref[k:k + 1, :]
    xcs = acc * jax.nn.sigmoid(acc)

    proj = jnp.dot(xcs.astype(jnp.bfloat16), xpw_ref[...],
                   preferred_element_type=_F32)
    dt = jnp.dot(proj[:, :DTR].astype(jnp.bfloat16), dtw_ref[...],
                 preferred_element_type=_F32) + dtb_ref[...]
    dt = jax.nn.softplus(dt)
    bm = proj[:, DTR:DTR + DS]
    cm = proj[:, DTR + DS:DTR + 2 * DS]

    pad_t = ((0, LP - L), (0, 0))
    dt_s[...] = jnp.pad(dt, pad_t)
    dtx_s[...] = jnp.pad(dt * xcs, pad_t)
    bm_s[...] = jnp.pad(bm, pad_t).reshape(NC, CH, DS)
    cm_s[...] = jnp.pad(cm, pad_t).reshape(NC, CH, DS)

    A = -jnp.exp(alog_ref[...])
    lane = jax.lax.broadcasted_iota(jnp.int32, (CH, CH * DS), 1)
    row = jax.lax.broadcasted_iota(jnp.int32, (CH, CH * DS), 0)
    cmask = (lane // DS) == row

    def chunk(c, hst):
        base = c * CH
        bT = jnp.transpose(bm_s[pl.ds(c, 1)][0])
        cc = cm_s[pl.ds(c, 1)][0]
        cblk = jnp.where(cmask, jnp.concatenate([cc] * CH, axis=1), 0.0)
        hs = []
        for s in range(CH):
            dtb = jnp.broadcast_to(dt_s[pl.ds(base + s, 1)], (DS, DI))
            dxb = jnp.broadcast_to(dtx_s[pl.ds(base + s, 1)], (DS, DI))
            a = jnp.exp(A * dtb)
            hst = a * hst + bT[:, s:s + 1] * dxb
            hs.append(hst)
        Hc = jnp.concatenate(hs, axis=0)
        Y = jnp.dot(cblk, Hc, preferred_element_type=_F32)
        ys_s[pl.ds(c, 1)] = Y.reshape(1, CH, DI)
        return hst

    jax.lax.fori_loop(0, NC, chunk, jnp.zeros((DS, DI), _F32), unroll=25)

    ys = ys_s[...].reshape(LP, DI)[:L, :]
    y = ys + xcs * dpar_ref[...]
    y = y * (z * jax.nn.sigmoid(z))
    out_ref[0] = jax.lax.dot_general(y.astype(jnp.bfloat16), ow_ref[...],
                                     _CONTRACT_LAST,
                                     preferred_element_type=_F32)


def _ln_body(h_ref, w_ref, b_ref, out_ref):
    h = h_ref[0]
    mu = jnp.mean(h, axis=1, keepdims=True)
    d = h - mu
    var = jnp.mean(d * d, axis=1, keepdims=True)
    out_ref[0] = d * jax.lax.rsqrt(var + 1e-5) * w_ref[...] + b_ref[...]


_CPARAMS = pltpu.CompilerParams(
    dimension_semantics=("parallel",),
    vmem_limit_bytes=100 * 1024 * 1024,
)


def _make_layer(Bz):
    return pl.pallas_call(
        _layer_body,
        grid=(Bz,),
        in_specs=[
            pl.BlockSpec((1, L, D), lambda b: (b, 0, 0)),
            pl.BlockSpec((2 * DI, D), lambda b: (0, 0)),
            pl.BlockSpec((DCONV, DI), lambda b: (0, 0)),
            pl.BlockSpec((1, DI), lambda b: (0, 0)),
            pl.BlockSpec((DI, DTR + 2 * DS), lambda b: (0, 0)),
            pl.BlockSpec((DTR, DI), lambda b: (0, 0)),
            pl.BlockSpec((1, DI), lambda b: (0, 0)),
            pl.BlockSpec((DS, DI), lambda b: (0, 0)),
            pl.BlockSpec((1, DI), lambda b: (0, 0)),
            pl.BlockSpec((D, DI), lambda b: (0, 0)),
        ],
        out_specs=pl.BlockSpec((1, L, D), lambda b: (b, 0, 0)),
        out_shape=jax.ShapeDtypeStruct((Bz, L, D), _F32),
        scratch_shapes=[
            pltpu.VMEM((DCONV - 1 + L, DI), _F32),
            pltpu.VMEM((LP, DI), _F32),
            pltpu.VMEM((LP, DI), _F32),
            pltpu.VMEM((NC, CH, DS), _F32),
            pltpu.VMEM((NC, CH, DS), _F32),
            pltpu.VMEM((NC, CH, DI), _F32),
        ],
        compiler_params=_CPARAMS,
    )


def _run(x, patch_w, patch_b, pos_embed, in_proj_w, conv_w, conv_b,
         x_proj_w, dt_proj_w, dt_proj_b, A_log, D_param, out_proj_w,
         norm_w, norm_b):
    Bz = x.shape[0]
    Hn = IMG // P
    xp = x.reshape(Bz, C, Hn, P, Hn, P).transpose(0, 2, 4, 1, 3, 5)
    xp = xp.reshape(Bz, N, C * P * P).astype(jnp.bfloat16)
    pw = patch_w.reshape(D, C * P * P).T.astype(jnp.bfloat16)
    pb = patch_b.reshape(1, D)
    inw16 = in_proj_w.astype(jnp.bfloat16)
    cwT = conv_w.transpose(0, 2, 1)
    cb2 = conv_b.reshape(NL, 1, DI)
    xpwT = x_proj_w.transpose(0, 2, 1).astype(jnp.bfloat16)
    dtwT = dt_proj_w.transpose(0, 2, 1).astype(jnp.bfloat16)
    dtb2 = dt_proj_b.reshape(NL, 1, DI)
    alogT = A_log.transpose(0, 2, 1)
    dpar2 = D_param.reshape(NL, 1, DI)
    ow16 = out_proj_w.astype(jnp.bfloat16)

    cparams = _CPARAMS

    h = pl.pallas_call(
        _embed_body,
        grid=(Bz,),
        in_specs=[
            pl.BlockSpec((1, N, C * P * P), lambda b: (b, 0, 0)),
            pl.BlockSpec((C * P * P, D), lambda b: (0, 0)),
            pl.BlockSpec((1, D), lambda b: (0, 0)),
            pl.BlockSpec((1, N, D), lambda b: (0, 0, 0)),
        ],
        out_specs=pl.BlockSpec((1, N, D), lambda b: (b, 0, 0)),
        out_shape=jax.ShapeDtypeStruct((Bz, N, D), _F32),
        compiler_params=cparams,
    )(xp, pw, pb, pos_embed)

    layer = _make_layer(Bz)

    for l in range(NL):
        h = layer(h, inw16[l], cwT[l], cb2[l], xpwT[l], dtwT[l], dtb2[l],
                  alogT[l], dpar2[l], ow16[l])

    out = pl.pallas_call(
        _ln_body,
        grid=(Bz,),
        in_specs=[
            pl.BlockSpec((1, L, D), lambda b: (b, 0, 0)),
            pl.BlockSpec((1, D), lambda b: (0, 0)),
            pl.BlockSpec((1, D), lambda b: (0, 0)),
        ],
        out_specs=pl.BlockSpec((1, L, D), lambda b: (b, 0, 0)),
        out_shape=jax.ShapeDtypeStruct((Bz, L, D), _F32),
        compiler_params=cparams,
    )(h, norm_w.reshape(1, D), norm_b.reshape(1, D))
    return out


def kernel(x, patch_w, patch_b, pos_embed, in_proj_w, conv_w, conv_b,
           x_proj_w, dt_proj_w, dt_proj_b, A_log, D_param, out_proj_w,
           norm_w, norm_b):
    return _run(x, patch_w, patch_b, pos_embed, in_proj_w, conv_w, conv_b,
                x_proj_w, dt_proj_w, dt_proj_b, A_log, D_param, out_proj_w,
                norm_w, norm_b)


# speedup vs baseline: 11.2686x; 1.1357x over previous
import jax
import jax.numpy as jnp
from jax.experimental import pallas as pl
from jax.experimental.pallas import tpu as pltpu

B = 8
C = 3
IMG = 224
P = 16
D = 768
NL = 12
DI = 2 * D
DS = 16
DTR = 48
N = (IMG // P) ** 2
DCONV = 4
L = N
CH = 8
NC = (L + CH - 1) // CH
LP = NC * CH

_F32 = jnp.float32


def _embed_body(xp_ref, pw_ref, pb_ref, pos_ref, out_ref):
    h = jnp.dot(xp_ref[0], pw_ref[...], preferred_element_type=_F32)
    out_ref[0] = h + pb_ref[...] + pos_ref[0]


_CONTRACT_LAST = (((1,), (1,)), ((), ()))


def _layer_body(h_ref, inw_ref, cw_ref, cb_ref, xpw_ref, dtw_ref, dtb_ref,
                alog_ref, dpar_ref, ow_ref, out_ref,
                cv_s, dt_s, dtx_s, bm_s, cm_s, ys_s):
    h = h_ref[0].astype(jnp.bfloat16)
    xz = jax.lax.dot_general(h, inw_ref[...], _CONTRACT_LAST,
                             preferred_element_type=_F32)
    xc = xz[:, :DI]
    z = xz[:, DI:]

    cv_s[:DCONV - 1, :] = jnp.zeros((DCONV - 1, DI), _F32)
    cv_s[DCONV - 1:, :] = xc
    acc = cb_ref[...]
    for k in range(DCONV):
        acc = acc + cv_s[k:k + L, :] * cw_ref[k:k + 1, :]
    xcs = acc * jax.nn.sigmoid(acc)

    proj = jnp.dot(xcs.astype(jnp.bfloat16), xpw_ref[...],
                   preferred_element_type=_F32)
    dt = jnp.dot(proj[:, :DTR].astype(jnp.bfloat16), dtw_ref[...],
                 preferred_element_type=_F32) + dtb_ref[...]
    dt = jax.nn.softplus(dt)
    bm = proj[:, DTR:DTR + DS]
    cm = proj[:, DTR + DS:DTR + 2 * DS]

    pad_t = ((0, LP - L), (0, 0))
    dt_s[...] = jnp.pad(dt, pad_t)
    dtx_s[...] = jnp.pad(dt * xcs, pad_t)
    bm_s[...] = jnp.pad(bm, pad_t).reshape(NC, CH, DS)
    cm_s[...] = jnp.pad(cm, pad_t).reshape(NC, CH, DS)

    A = -jnp.exp(alog_ref[...]) * 1.4426950408889634
    lane = jax.lax.broadcasted_iota(jnp.int32, (CH, CH * DS), 1)
    row = jax.lax.broadcasted_iota(jnp.int32, (CH, CH * DS), 0)
    cmask = (lane // DS) == row

    def chunk(c, hst):
        base = c * CH
        bT = jnp.transpose(bm_s[pl.ds(c, 1)][0])
        cc = cm_s[pl.ds(c, 1)][0]
        cblk = jnp.where(cmask, jnp.concatenate([cc] * CH, axis=1), 0.0)
        hs = []
        for s in range(CH):
            dtb = dt_s[pl.ds(base + s, DS, 0)]
            dxb = dtx_s[pl.ds(base + s, DS, 0)]
            a = jnp.exp2(A * dtb)
            hst = a * hst + bT[:, s:s + 1] * dxb
            hs.append(hst)
        Hc = jnp.concatenate(hs, axis=0)
        Y = jnp.dot(cblk, Hc, preferred_element_type=_F32)
        ys_s[pl.ds(c, 1)] = Y.reshape(1, CH, DI)
        return hst

    jax.lax.fori_loop(0, NC, chunk, jnp.zeros((DS, DI), _F32), unroll=25)

    ys = ys_s[...].reshape(LP, DI)[:L, :]
    y = ys + xcs * dpar_ref[...]
    y = y * (z * jax.nn.sigmoid(z))
    out_ref[0] = jax.lax.dot_general(y.astype(jnp.bfloat16), ow_ref[...],
                                     _CONTRACT_LAST,
                                     preferred_element_type=_F32)


def _ln_body(h_ref, w_ref, b_ref, out_ref):
    h = h_ref[0]
    mu = jnp.mean(h, axis=1, keepdims=True)
    d = h - mu
    var = jnp.mean(d * d, axis=1, keepdims=True)
    out_ref[0] = d * jax.lax.rsqrt(var + 1e-5) * w_ref[...] + b_ref[...]


_CPARAMS = pltpu.CompilerParams(
    dimension_semantics=("parallel",),
    vmem_limit_bytes=100 * 1024 * 1024,
)


def _make_layer(Bz):
    return pl.pallas_call(
        _layer_body,
        grid=(Bz,),
        in_specs=[
            pl.BlockSpec((1, L, D), lambda b: (b, 0, 0)),
            pl.BlockSpec((2 * DI, D), lambda b: (0, 0)),
            pl.BlockSpec((DCONV, DI), lambda b: (0, 0)),
            pl.BlockSpec((1, DI), lambda b: (0, 0)),
            pl.BlockSpec((DI, DTR + 2 * DS), lambda b: (0, 0)),
            pl.BlockSpec((DTR, DI), lambda b: (0, 0)),
            pl.BlockSpec((1, DI), lambda b: (0, 0)),
            pl.BlockSpec((DS, DI), lambda b: (0, 0)),
            pl.BlockSpec((1, DI), lambda b: (0, 0)),
            pl.BlockSpec((D, DI), lambda b: (0, 0)),
        ],
        out_specs=pl.BlockSpec((1, L, D), lambda b: (b, 0, 0)),
        out_shape=jax.ShapeDtypeStruct((Bz, L, D), _F32),
        scratch_shapes=[
            pltpu.VMEM((DCONV - 1 + L, DI), _F32),
            pltpu.VMEM((LP, DI), _F32),
            pltpu.VMEM((LP, DI), _F32),
            pltpu.VMEM((NC, CH, DS), _F32),
            pltpu.VMEM((NC, CH, DS), _F32),
            pltpu.VMEM((NC, CH, DI), _F32),
        ],
        compiler_params=_CPARAMS,
    )


def _run(x, patch_w, patch_b, pos_embed, in_proj_w, conv_w, conv_b,
         x_proj_w, dt_proj_w, dt_proj_b, A_log, D_param, out_proj_w,
         norm_w, norm_b):
    Bz = x.shape[0]
    Hn = IMG // P
    xp = x.reshape(Bz, C, Hn, P, Hn, P).transpose(0, 2, 4, 1, 3, 5)
    xp = xp.reshape(Bz, N, C * P * P).astype(jnp.bfloat16)
    pw = patch_w.reshape(D, C * P * P).T.astype(jnp.bfloat16)
    pb = patch_b.reshape(1, D)
    inw16 = in_proj_w.astype(jnp.bfloat16)
    cwT = conv_w.transpose(0, 2, 1)
    cb2 = conv_b.reshape(NL, 1, DI)
    xpwT = x_proj_w.transpose(0, 2, 1).astype(jnp.bfloat16)
    dtwT = dt_proj_w.transpose(0, 2, 1).astype(jnp.bfloat16)
    dtb2 = dt_proj_b.reshape(NL, 1, DI)
    alogT = A_log.transpose(0, 2, 1)
    dpar2 = D_param.reshape(NL, 1, DI)
    ow16 = out_proj_w.astype(jnp.bfloat16)

    cparams = _CPARAMS

    h = pl.pallas_call(
        _embed_body,
        grid=(Bz,),
        in_specs=[
            pl.BlockSpec((1, N, C * P * P), lambda b: (b, 0, 0)),
            pl.BlockSpec((C * P * P, D), lambda b: (0, 0)),
            pl.BlockSpec((1, D), lambda b: (0, 0)),
            pl.BlockSpec((1, N, D), lambda b: (0, 0, 0)),
        ],
        out_specs=pl.BlockSpec((1, N, D), lambda b: (b, 0, 0)),
        out_shape=jax.ShapeDtypeStruct((Bz, N, D), _F32),
        compiler_params=cparams,
    )(xp, pw, pb, pos_embed)

    layer = _make_layer(Bz)

    for l in range(NL):
        h = layer(h, inw16[l], cwT[l], cb2[l], xpwT[l], dtwT[l], dtb2[l],
                  alogT[l], dpar2[l], ow16[l])

    out = pl.pallas_call(
        _ln_body,
        grid=(Bz,),
        in_specs=[
            pl.BlockSpec((1, L, D), lambda b: (b, 0, 0)),
            pl.BlockSpec((1, D), lambda b: (0, 0)),
            pl.BlockSpec((1, D), lambda b: (0, 0)),
        ],
        out_specs=pl.BlockSpec((1, L, D), lambda b: (b, 0, 0)),
        out_shape=jax.ShapeDtypeStruct((Bz, L, D), _F32),
        compiler_params=cparams,
    )(h, norm_w.reshape(1, D), norm_b.reshape(1, D))
    return out


def kernel(x, patch_w, patch_b, pos_embed, in_proj_w, conv_w, conv_b,
           x_proj_w, dt_proj_w, dt_proj_b, A_log, D_param, out_proj_w,
           norm_w, norm_b):
    return _run(x, patch_w, patch_b, pos_embed, in_proj_w, conv_w, conv_b,
                x_proj_w, dt_proj_w, dt_proj_b, A_log, D_param, out_proj_w,
                norm_w, norm_b)
